# baseline (speedup 1.0000x reference)
import functools

import jax
import jax.numpy as jnp
from jax import lax
from jax.experimental import pallas as pl
from jax.experimental.pallas import tpu as pltpu
from jax.experimental.pallas import tpu_sc as plsc

NC = 2
NS = 16
NW = NC * NS
LANES = 16

CHUNK = 32
IDX_MINOR = 112
IDX_MINOR_T = 128


def _sc_tgather(b, d, rows_per_w):
  mesh = plsc.VectorSubcoreMesh(core_axis_name="c", subcore_axis_name="s")
  n_gath = rows_per_w // IDX_MINOR_T

  @functools.partial(
      pl.kernel,
      out_type=jax.ShapeDtypeStruct((b, d), jnp.float32),
      mesh=mesh,
      compiler_params=pltpu.CompilerParams(
          needs_layout_passes=False, use_tc_tiling_on_sc=False),
      scratch_types=[
          pltpu.VMEM((rows_per_w,), jnp.int32),
          pltpu.VMEM((rows_per_w, d), jnp.float32),
          pltpu.SemaphoreType.DMA,
      ],
  )
  def tg_kernel(tidx_hbm, in_emb_hbm, tvec_hbm, tidx_v, t_v, sem):
    wid = lax.axis_index("s") * NC + lax.axis_index("c")
    base = wid * rows_per_w
    pltpu.sync_copy(tidx_hbm.at[pl.ds(base, rows_per_w)], tidx_v)
    for j in range(n_gath):
      pltpu.async_copy(
          in_emb_hbm.at[tidx_v.at[pl.ds(j * IDX_MINOR_T, IDX_MINOR_T)]],
          t_v.at[pl.ds(j * IDX_MINOR_T, IDX_MINOR_T)],
          sem,
      )
    for j in range(n_gath):
      pltpu.make_async_copy(
          in_emb_hbm.at[tidx_v.at[pl.ds(j * IDX_MINOR_T, IDX_MINOR_T)]],
          t_v.at[pl.ds(j * IDX_MINOR_T, IDX_MINOR_T)],
          sem,
      ).wait()
    pltpu.sync_copy(t_v, tvec_hbm.at[pl.ds(base, rows_per_w)])

  return tg_kernel


def _sc_scores(b, k1, d, rows_per_w, nchunk):
  n_gath = (CHUNK * k1) // IDX_MINOR
  assert nchunk % 2 == 0 and nchunk >= 4

  mesh = plsc.VectorSubcoreMesh(core_axis_name="c", subcore_axis_name="s")

  @functools.partial(
      pl.kernel,
      out_type=jax.ShapeDtypeStruct((b, 2 * LANES), jnp.float32),
      mesh=mesh,
      compiler_params=pltpu.CompilerParams(
          needs_layout_passes=False, use_tc_tiling_on_sc=False),
      scratch_types=[
          pltpu.VMEM((2, CHUNK * k1), jnp.int32),
          pltpu.VMEM((2, CHUNK, d), jnp.float32),
          pltpu.VMEM((2, CHUNK * k1, d), jnp.float32),
          pltpu.VMEM((CHUNK, 2 * LANES), jnp.float32),
          pltpu.SemaphoreType.DMA,
          pltpu.SemaphoreType.DMA,
      ],
  )
  def sc_kernel(aidx_hbm, tvec_hbm, out_emb_hbm, scores_hbm,
                aidx_v, t_v, a_v, sc_v, sem0, sem1):
    wid = lax.axis_index("s") * NC + lax.axis_index("c")
    lane = lax.iota(jnp.int32, LANES)
    sems = (sem0, sem1)

    def issue(i, s):
      base = wid * rows_per_w + i * CHUNK
      sem = sems[s]
      pltpu.sync_copy(aidx_hbm.at[pl.ds(base * k1, CHUNK * k1)],
                      aidx_v.at[s])
      pltpu.async_copy(tvec_hbm.at[pl.ds(base, CHUNK)], t_v.at[s], sem)
      for j in range(n_gath):
        pltpu.async_copy(
            out_emb_hbm.at[aidx_v.at[s].at[pl.ds(j * IDX_MINOR, IDX_MINOR)]],
            a_v.at[s].at[pl.ds(j * IDX_MINOR, IDX_MINOR)],
            sem,
        )

    def wait(s):
      sem = sems[s]
      pltpu.make_async_copy(tvec_hbm.at[pl.ds(0, CHUNK)], t_v.at[s],
                            sem).wait()
      for j in range(n_gath):
        pltpu.make_async_copy(
            out_emb_hbm.at[aidx_v.at[s].at[pl.ds(j * IDX_MINOR, IDX_MINOR)]],
            a_v.at[s].at[pl.ds(j * IDX_MINOR, IDX_MINOR)],
            sem,
        ).wait()

    def compute(i, s):
      base = wid * rows_per_w + i * CHUNK

      def row_body(r, _):
        t0 = t_v[s, r, pl.ds(0, LANES)]
        t1 = t_v[s, r, pl.ds(LANES, LANES)]
        t2 = t_v[s, r, pl.ds(2 * LANES, LANES)]
        t3 = t_v[s, r, pl.ds(3 * LANES, LANES)]
        acc_a = jnp.zeros((LANES,), jnp.float32)
        acc_b = jnp.zeros((LANES,), jnp.float32)
        for k in range(k1):
          row = r * k1 + k
          p = t0 * a_v[s, row, pl.ds(0, LANES)]
          p += t1 * a_v[s, row, pl.ds(LANES, LANES)]
          p += t2 * a_v[s, row, pl.ds(2 * LANES, LANES)]
          p += t3 * a_v[s, row, pl.ds(3 * LANES, LANES)]
          t = plsc.cumsum(p)[jnp.full((LANES,), LANES - 1, jnp.int32)]
          if k < LANES:
            acc_a = jnp.where(lane == k, t, acc_a)
          else:
            acc_b = jnp.where(lane == (k - LANES), t, acc_b)
        sc_v[r, pl.ds(0, LANES)] = acc_a
        sc_v[r, pl.ds(LANES, LANES)] = acc_b
        return 0

      lax.fori_loop(0, CHUNK, row_body, 0)
      pltpu.sync_copy(sc_v, scores_hbm.at[pl.ds(base, CHUNK)])

    issue(0, 0)
    issue(1, 1)

    def pair_body(g, _):
      i = 2 * g
      wait(0)
      issue_i = i + 2
      compute(i, 0)
      issue(issue_i, 0)
      wait(1)
      compute(i + 1, 1)
      issue(issue_i + 1, 1)
      return 0

    lax.fori_loop(0, nchunk // 2 - 1, pair_body, 0)
    wait(0)
    compute(nchunk - 2, 0)
    wait(1)
    compute(nchunk - 1, 1)

  return sc_kernel


def _loss_body(nk, b, s_ref, o_ref):
  s = s_ref[...]
  col = lax.broadcasted_iota(jnp.int32, s.shape, 1)
  x = jnp.where(col == nk, -s, s)
  sp = jnp.maximum(x, 0.0) + jnp.log1p(jnp.exp(-jnp.abs(x)))
  sp = jnp.where(col <= nk, sp, 0.0)
  o_ref[...] = (jnp.sum(sp) / b).reshape(1, 1)


def kernel(target, context, negative_samples, in_emb, out_emb):
  b, k = negative_samples.shape
  d = in_emb.shape[1]
  k1 = k + 1
  rows_per_w = b // NW
  nchunk = rows_per_w // CHUNK

  idx_all = jnp.concatenate([negative_samples, context[:, None]], axis=1)
  idx_all = idx_all.reshape(b * k1)

  tvecs = _sc_tgather(b, d, rows_per_w)(target, in_emb)
  scores = _sc_scores(b, k1, d, rows_per_w, nchunk)(
      idx_all, tvecs, out_emb)

  loss = pl.pallas_call(
      functools.partial(_loss_body, k, b),
      out_shape=jax.ShapeDtypeStruct((1, 1), jnp.float32),
  )(scores)
  return loss[0, 0]


# speedup vs baseline: 1.0074x; 1.0074x over previous
import functools

import jax
import jax.numpy as jnp
from jax import lax
from jax.experimental import pallas as pl
from jax.experimental.pallas import tpu as pltpu
from jax.experimental.pallas import tpu_sc as plsc

NC = 2
NS = 16
NW = NC * NS
LANES = 16

CHUNK = 32
IDX_MINOR = 112


def _sc_scores(b, k1, d, rows_per_w, nchunk):
  n_gath = (CHUNK * k1) // IDX_MINOR
  assert nchunk % 2 == 0 and nchunk >= 4

  mesh = plsc.VectorSubcoreMesh(core_axis_name="c", subcore_axis_name="s")

  @functools.partial(
      pl.kernel,
      out_type=jax.ShapeDtypeStruct((b, 2 * LANES), jnp.float32),
      mesh=mesh,
      compiler_params=pltpu.CompilerParams(
          needs_layout_passes=False, use_tc_tiling_on_sc=False),
      scratch_types=[
          pltpu.VMEM((2, CHUNK), jnp.int32),
          pltpu.VMEM((2, CHUNK * k1), jnp.int32),
          pltpu.VMEM((2, CHUNK, d), jnp.float32),
          pltpu.VMEM((2, CHUNK * k1, d), jnp.float32),
          pltpu.VMEM((CHUNK, 2 * LANES), jnp.float32),
          pltpu.SemaphoreType.DMA,
          pltpu.SemaphoreType.DMA,
      ],
  )
  def sc_kernel(tidx_hbm, aidx_hbm, in_emb_hbm, out_emb_hbm, scores_hbm,
                tidx_v, aidx_v, t_v, a_v, sc_v, sem0, sem1):
    wid = lax.axis_index("s") * NC + lax.axis_index("c")
    lane = lax.iota(jnp.int32, LANES)
    sems = (sem0, sem1)

    def issue(i, s):
      base = wid * rows_per_w + i * CHUNK
      sem = sems[s]
      pltpu.sync_copy(tidx_hbm.at[pl.ds(base, CHUNK)], tidx_v.at[s])
      pltpu.sync_copy(aidx_hbm.at[pl.ds(base * k1, CHUNK * k1)],
                      aidx_v.at[s])
      pltpu.async_copy(in_emb_hbm.at[tidx_v.at[s]], t_v.at[s], sem)
      for j in range(n_gath):
        pltpu.async_copy(
            out_emb_hbm.at[aidx_v.at[s].at[pl.ds(j * IDX_MINOR, IDX_MINOR)]],
            a_v.at[s].at[pl.ds(j * IDX_MINOR, IDX_MINOR)],
            sem,
        )

    def wait(s):
      sem = sems[s]
      pltpu.make_async_copy(in_emb_hbm.at[tidx_v.at[s]], t_v.at[s],
                            sem).wait()
      for j in range(n_gath):
        pltpu.make_async_copy(
            out_emb_hbm.at[aidx_v.at[s].at[pl.ds(j * IDX_MINOR, IDX_MINOR)]],
            a_v.at[s].at[pl.ds(j * IDX_MINOR, IDX_MINOR)],
            sem,
        ).wait()

    def compute(i, s):
      base = wid * rows_per_w + i * CHUNK

      def row_body(r, _):
        t0 = t_v[s, r, pl.ds(0, LANES)]
        t1 = t_v[s, r, pl.ds(LANES, LANES)]
        t2 = t_v[s, r, pl.ds(2 * LANES, LANES)]
        t3 = t_v[s, r, pl.ds(3 * LANES, LANES)]
        acc_a = jnp.zeros((LANES,), jnp.float32)
        acc_b = jnp.zeros((LANES,), jnp.float32)
        for k in range(k1):
          row = r * k1 + k
          p = t0 * a_v[s, row, pl.ds(0, LANES)]
          p += t1 * a_v[s, row, pl.ds(LANES, LANES)]
          p += t2 * a_v[s, row, pl.ds(2 * LANES, LANES)]
          p += t3 * a_v[s, row, pl.ds(3 * LANES, LANES)]
          t = plsc.cumsum(p)[jnp.full((LANES,), LANES - 1, jnp.int32)]
          if k < LANES:
            acc_a = jnp.where(lane == k, t, acc_a)
          else:
            acc_b = jnp.where(lane == (k - LANES), t, acc_b)
        sc_v[r, pl.ds(0, LANES)] = acc_a
        sc_v[r, pl.ds(LANES, LANES)] = acc_b
        return 0

      lax.fori_loop(0, CHUNK, row_body, 0)
      pltpu.sync_copy(sc_v, scores_hbm.at[pl.ds(base, CHUNK)])

    issue(0, 0)
    issue(1, 1)

    def pair_body(g, _):
      i = 2 * g
      wait(0)
      issue_i = i + 2
      compute(i, 0)
      issue(issue_i, 0)
      wait(1)
      compute(i + 1, 1)
      issue(issue_i + 1, 1)
      return 0

    lax.fori_loop(0, nchunk // 2 - 1, pair_body, 0)
    wait(0)
    compute(nchunk - 2, 0)
    wait(1)
    compute(nchunk - 1, 1)

  return sc_kernel


def _loss_body(nk, b, s_ref, o_ref):
  s = s_ref[...]
  col = lax.broadcasted_iota(jnp.int32, s.shape, 1)
  x = jnp.where(col == nk, -s, s)
  sp = jnp.maximum(x, 0.0) + jnp.log1p(jnp.exp(-jnp.abs(x)))
  sp = jnp.where(col <= nk, sp, 0.0)
  o_ref[...] = (jnp.sum(sp) / b).reshape(1, 1)


def kernel(target, context, negative_samples, in_emb, out_emb):
  b, k = negative_samples.shape
  d = in_emb.shape[1]
  k1 = k + 1
  rows_per_w = b // NW
  nchunk = rows_per_w // CHUNK

  idx_all = jnp.concatenate([negative_samples, context[:, None]], axis=1)
  idx_all = idx_all.reshape(b * k1)

  scores = _sc_scores(b, k1, d, rows_per_w, nchunk)(
      target, idx_all, in_emb, out_emb)

  loss = pl.pallas_call(
      functools.partial(_loss_body, k, b),
      out_shape=jax.ShapeDtypeStruct((1, 1), jnp.float32),
  )(scores)
  return loss[0, 0]
